# baseline (speedup 1.0000x reference)
import functools

import jax
import jax.numpy as jnp
from jax import lax
from jax.experimental import pallas as pl
from jax.experimental.pallas import tpu as pltpu
from jax.experimental.pallas import tpu_sc as plsc

B = 16384
K = 64
NC = 2
NS = 16
NW = NC * NS
BPW = B // NW
NG = BPW // 16
EPS = 1e-6


def _sqrt16(x):
    i = plsc.bitcast(x, jnp.int32)
    i = jnp.int32(0x5F3759DF) - lax.shift_right_arithmetic(i, 1)
    y = plsc.bitcast(i, jnp.float32)
    xh = x * 0.5
    for _ in range(3):
        y = y * (1.5 - xh * y * y)
    return x * y


def _cris_body(user_hbm, item_hbm, ebd_u_hbm, ebd_i_hbm, proto_hbm,
               c_out, i_out,
               idx_u, idx_i, bu, bi, proto_v, c_v, i_v, *sems):
    cid = lax.axis_index("c")
    sid = lax.axis_index("s")
    wid = sid * NC + cid
    base = wid * BPW

    for ch in range(4):
        pltpu.sync_copy(user_hbm.at[wid * 4 + ch],
                        idx_u.at[pl.ds(ch * 128, 128)])
        pltpu.sync_copy(item_hbm.at[wid * 4 + ch],
                        idx_i.at[pl.ds(ch * 128, 128)])
    pltpu.sync_copy(proto_hbm, proto_v)

    lane = lax.broadcasted_iota(jnp.int32, (16,), 0)

    for j in range(2 * K // 16):
        proto_v[pl.ds(j * 16, 16)] = proto_v[pl.ds(j * 16, 16)] + EPS

    def fetch(g, buf):
        iv_u = idx_u[pl.ds(g * 16, 16)]
        iv_i = idx_i[pl.ds(g * 16, 16)]
        for l in range(16):
            pltpu.async_copy(ebd_u_hbm.at[pl.ds(iv_u[l], 1)],
                             bu.at[pl.ds(buf * 16 + l, 1)], sems[l % 4])
            pltpu.async_copy(ebd_i_hbm.at[pl.ds(iv_i[l], 1)],
                             bi.at[pl.ds(buf * 16 + l, 1)], sems[4 + l % 4])

    def drain(buf):
        for q in range(4):
            pltpu.make_async_copy(
                ebd_u_hbm.at[pl.ds(0, 4)],
                bu.at[pl.ds(buf * 16 + q * 4, 4)], sems[q]).wait()
            pltpu.make_async_copy(
                ebd_i_hbm.at[pl.ds(0, 4)],
                bi.at[pl.ds(buf * 16 + q * 4, 4)], sems[4 + q]).wait()

    def compute(g, buf):
        def k_body(k, carry):
            s0, s1 = carry
            kvec = jnp.full((16,), k, dtype=jnp.int32)
            rvec = buf * 16 + lane
            iu = plsc.load_gather(bu, [rvec, kvec])
            ii = plsc.load_gather(bi, [rvec, kvec])
            q0 = plsc.load_gather(proto_v, [kvec])
            q1 = plsc.load_gather(proto_v, [kvec + K])
            ui = iu + ii
            t0 = q0 - ui
            t1 = q1 - ui
            return (s0 + t0 * t0, s1 + t1 * t1)

        z = jnp.zeros((16,), jnp.float32)
        s0, s1 = lax.fori_loop(0, K, k_body, (z, z))
        row0 = g * 16
        c_v[pl.ds(row0, 16)] = _sqrt16(s0)
        i_v[pl.ds(row0, 16)] = _sqrt16(s1)

    fetch(0, 0)

    def group_body(g, _):
        buf = lax.rem(g, 2)
        nbuf = lax.rem(g + 1, 2)

        @pl.when(g + 1 < NG)
        def _():
            fetch(g + 1, nbuf)

        drain(buf)
        compute(g, buf)
        return 0

    lax.fori_loop(0, NG, group_body, 0)

    pltpu.sync_copy(c_v, c_out.at[pl.ds(base, BPW)])
    pltpu.sync_copy(i_v, i_out.at[pl.ds(base, BPW)])


_cris = functools.partial(
    pl.kernel,
    mesh=plsc.VectorSubcoreMesh(core_axis_name="c", subcore_axis_name="s"),
    compiler_params=pltpu.CompilerParams(needs_layout_passes=False),
    out_type=[
        jax.ShapeDtypeStruct((B,), jnp.float32),
        jax.ShapeDtypeStruct((B,), jnp.float32),
    ],
    scratch_types=[
        pltpu.VMEM((BPW,), jnp.int32),
        pltpu.VMEM((BPW,), jnp.int32),
        pltpu.VMEM((2 * 16, K), jnp.float32),
        pltpu.VMEM((2 * 16, K), jnp.float32),
        pltpu.VMEM((2 * K,), jnp.float32),
        pltpu.VMEM((BPW,), jnp.float32),
        pltpu.VMEM((BPW,), jnp.float32),
        pltpu.SemaphoreType.DMA,
        pltpu.SemaphoreType.DMA,
        pltpu.SemaphoreType.DMA,
        pltpu.SemaphoreType.DMA,
        pltpu.SemaphoreType.DMA,
        pltpu.SemaphoreType.DMA,
        pltpu.SemaphoreType.DMA,
        pltpu.SemaphoreType.DMA,
    ],
)(_cris_body)


@jax.jit
def kernel(user, item, ebd_user, ebd_item, ebd_prototype):
    user2d = user.astype(jnp.int32).reshape(NW * 4, 128)
    item2d = item.astype(jnp.int32).reshape(NW * 4, 128)
    proto_flat = ebd_prototype.reshape(2 * K)
    c_dist, i_dist = _cris(user2d, item2d, ebd_user, ebd_item, proto_flat)
    return c_dist, i_dist


# speedup vs baseline: 1.0053x; 1.0053x over previous
import functools

import jax
import jax.numpy as jnp
from jax import lax
from jax.experimental import pallas as pl
from jax.experimental.pallas import tpu as pltpu
from jax.experimental.pallas import tpu_sc as plsc

B = 16384
K = 64
NC = 2
NS = 16
NW = NC * NS
BPW = B // NW
NG = BPW // 16
EPS = 1e-6


def _sqrt16(x):
    i = plsc.bitcast(x, jnp.int32)
    i = jnp.int32(0x5F3759DF) - lax.shift_right_arithmetic(i, 1)
    y = plsc.bitcast(i, jnp.float32)
    xh = x * 0.5
    for _ in range(3):
        y = y * (1.5 - xh * y * y)
    return x * y


def _cris_body(user_hbm, item_hbm, ebd_u_hbm, ebd_i_hbm, proto_hbm,
               c_out, i_out,
               idx_u, idx_i, bu, bi, proto_v, c_v, i_v, sem_u, sem_i):
    cid = lax.axis_index("c")
    sid = lax.axis_index("s")
    wid = sid * NC + cid
    base = wid * BPW

    for ch in range(4):
        pltpu.sync_copy(user_hbm.at[wid * 4 + ch],
                        idx_u.at[pl.ds(ch * 128, 128)])
        pltpu.sync_copy(item_hbm.at[wid * 4 + ch],
                        idx_i.at[pl.ds(ch * 128, 128)])
    pltpu.sync_copy(proto_hbm, proto_v)

    lane = lax.broadcasted_iota(jnp.int32, (16,), 0)

    for j in range(2 * K // 16):
        proto_v[pl.ds(j * 16, 16)] = proto_v[pl.ds(j * 16, 16)] + EPS

    def fetch(g, buf):
        iv_u = idx_u[pl.ds(g * 16, 16)]
        iv_i = idx_i[pl.ds(g * 16, 16)]
        for l in range(16):
            pltpu.async_copy(ebd_u_hbm.at[pl.ds(iv_u[l], 1)],
                             bu.at[pl.ds(buf * 16 + l, 1)], sem_u)
            pltpu.async_copy(ebd_i_hbm.at[pl.ds(iv_i[l], 1)],
                             bi.at[pl.ds(buf * 16 + l, 1)], sem_i)

    def drain(buf):
        pltpu.make_async_copy(
            ebd_u_hbm.at[pl.ds(0, 16)], bu.at[pl.ds(buf * 16, 16)], sem_u).wait()
        pltpu.make_async_copy(
            ebd_i_hbm.at[pl.ds(0, 16)], bi.at[pl.ds(buf * 16, 16)], sem_i).wait()

    def compute(g, buf):
        def k_body(k, carry):
            s0, s1 = carry
            kvec = jnp.full((16,), k, dtype=jnp.int32)
            rvec = buf * 16 + lane
            iu = plsc.load_gather(bu, [rvec, kvec])
            ii = plsc.load_gather(bi, [rvec, kvec])
            q0 = plsc.load_gather(proto_v, [kvec])
            q1 = plsc.load_gather(proto_v, [kvec + K])
            ui = iu + ii
            t0 = q0 - ui
            t1 = q1 - ui
            return (s0 + t0 * t0, s1 + t1 * t1)

        z = jnp.zeros((16,), jnp.float32)
        s0, s1 = lax.fori_loop(0, K, k_body, (z, z))
        row0 = g * 16
        c_v[pl.ds(row0, 16)] = _sqrt16(s0)
        i_v[pl.ds(row0, 16)] = _sqrt16(s1)

    for g0 in range(3):
        fetch(g0, g0)

    def group_body(g, _):
        buf = lax.rem(g, 4)

        @pl.when(g + 3 < NG)
        def _():
            fetch(g + 3, lax.rem(g + 3, 4))

        drain(buf)
        compute(g, buf)
        return 0

    lax.fori_loop(0, NG, group_body, 0)

    pltpu.sync_copy(c_v, c_out.at[pl.ds(base, BPW)])
    pltpu.sync_copy(i_v, i_out.at[pl.ds(base, BPW)])


_cris = functools.partial(
    pl.kernel,
    mesh=plsc.VectorSubcoreMesh(core_axis_name="c", subcore_axis_name="s"),
    compiler_params=pltpu.CompilerParams(needs_layout_passes=False),
    out_type=[
        jax.ShapeDtypeStruct((B,), jnp.float32),
        jax.ShapeDtypeStruct((B,), jnp.float32),
    ],
    scratch_types=[
        pltpu.VMEM((BPW,), jnp.int32),
        pltpu.VMEM((BPW,), jnp.int32),
        pltpu.VMEM((4 * 16, K), jnp.float32),
        pltpu.VMEM((4 * 16, K), jnp.float32),
        pltpu.VMEM((2 * K,), jnp.float32),
        pltpu.VMEM((BPW,), jnp.float32),
        pltpu.VMEM((BPW,), jnp.float32),
        pltpu.SemaphoreType.DMA,
        pltpu.SemaphoreType.DMA,
    ],
)(_cris_body)


@jax.jit
def kernel(user, item, ebd_user, ebd_item, ebd_prototype):
    user2d = user.astype(jnp.int32).reshape(NW * 4, 128)
    item2d = item.astype(jnp.int32).reshape(NW * 4, 128)
    proto_flat = ebd_prototype.reshape(2 * K)
    c_dist, i_dist = _cris(user2d, item2d, ebd_user, ebd_item, proto_flat)
    return c_dist, i_dist
